# baseline (speedup 1.0000x reference)
import jax
import jax.numpy as jnp
import numpy as np
from jax import lax
from jax.experimental import pallas as pl
from jax.experimental.pallas import tpu as pltpu
from jax.experimental.pallas import tpu_sc as plsc

N_LEVELS = 12
F_PER = 2
LOG2_T = 19
T = 1 << LOG2_T
BASE_RES = 16
PER_LEVEL_SCALE = 2.0
N_POINTS = 524288

P1 = int(np.uint32(2654435761).view(np.int32))
P2 = int(np.uint32(805459861).view(np.int32))

NC, NS = 2, 16
NW = NC * NS
P_PER_W = N_POINTS // NW
C = 128
N_CHUNKS = P_PER_W // C
E_PER_F = N_LEVELS * 8 * C
NF = N_LEVELS * F_PER
E_OUT = NF * C

_RES = [int(np.ceil(BASE_RES * (PER_LEVEL_SCALE ** l))) for l in range(N_LEVELS)]
_DENSE = [(r + 1) ** 3 <= T for r in _RES]


def _index_pass(x_v, xb, i0_v, i1_v):
    for l in range(N_LEVELS):
        res = _RES[l]
        lT2 = l * T

        def idx_g(g, _, l=l, res=res, lT2=lT2):
            px = x_v[pl.ds(xb + 0 * C + g * 16, 16)] * jnp.float32(res)
            py = x_v[pl.ds(xb + 1 * C + g * 16, 16)] * jnp.float32(res)
            pz = x_v[pl.ds(xb + 2 * C + g * 16, 16)] * jnp.float32(res)
            ix = px.astype(jnp.int32)
            iy = py.astype(jnp.int32)
            iz = pz.astype(jnp.int32)
            if _DENSE[l]:
                s = res + 1
                y0 = iy * jnp.int32(s)
                y1 = y0 + jnp.int32(s)
                z0 = iz * jnp.int32(s * s)
                z1 = z0 + jnp.int32(s * s)
                x0 = ix + jnp.int32(lT2)
                x1 = x0 + jnp.int32(1)
                for c in range(8):
                    h = ((x1 if (c & 1) else x0)
                         + (y1 if (c & 2) else y0)
                         + (z1 if (c & 4) else z0))
                    e = h + (h & jnp.int32(-128))
                    off = (l * 8 + c) * C
                    i0_v[pl.ds(off + g * 16, 16)] = e
                    i1_v[pl.ds(off + g * 16, 16)] = e + jnp.int32(128)
            else:
                y0 = iy * jnp.int32(P1)
                y1 = y0 + jnp.int32(P1)
                z0 = iz * jnp.int32(P2)
                z1 = z0 + jnp.int32(P2)
                x1 = ix + jnp.int32(1)
                for c in range(8):
                    h = (((x1 if (c & 1) else ix)
                          ^ (y1 if (c & 2) else y0)
                          ^ (z1 if (c & 4) else z0)) & jnp.int32(T - 1)
                         ) + jnp.int32(lT2)
                    e = h + (h & jnp.int32(-128))
                    off = (l * 8 + c) * C
                    i0_v[pl.ds(off + g * 16, 16)] = e
                    i1_v[pl.ds(off + g * 16, 16)] = e + jnp.int32(128)
            return 0

        lax.fori_loop(0, C // 16, idx_g, 0, unroll=False)


def _acc_pass(x_v, xb, f0_v, f1_v, out_v):
    for l in range(N_LEVELS):
        res = _RES[l]

        def acc_g(g, _, l=l, res=res):
            px = x_v[pl.ds(xb + 0 * C + g * 16, 16)] * jnp.float32(res)
            py = x_v[pl.ds(xb + 1 * C + g * 16, 16)] * jnp.float32(res)
            pz = x_v[pl.ds(xb + 2 * C + g * 16, 16)] * jnp.float32(res)
            wx = px - px.astype(jnp.int32).astype(jnp.float32)
            wy = py - py.astype(jnp.int32).astype(jnp.float32)
            wz = pz - pz.astype(jnp.int32).astype(jnp.float32)
            ox = jnp.float32(1.0) - wx
            oy = jnp.float32(1.0) - wy
            oz = jnp.float32(1.0) - wz
            wxy = (ox * oy, wx * oy, ox * wy, wx * wy)
            w8 = [wxy[c & 3] * (wz if (c & 4) else oz) for c in range(8)]
            base = (l * 8) * C + g * 16
            t0 = [None] * 8
            t1 = [None] * 8
            for c in range(8):
                f0 = f0_v[pl.ds(base + c * C, 16)]
                f1 = f1_v[pl.ds(base + c * C, 16)]
                t0[c] = f0 * w8[c]
                t1[c] = f1 * w8[c]
            acc0 = ((t0[0] + t0[1]) + (t0[2] + t0[3])) + (
                (t0[4] + t0[5]) + (t0[6] + t0[7]))
            acc1 = ((t1[0] + t1[1]) + (t1[2] + t1[3])) + (
                (t1[4] + t1[5]) + (t1[6] + t1[7]))
            out_v[pl.ds((2 * l) * C + g * 16, 16)] = acc0
            out_v[pl.ds((2 * l + 1) * C + g * 16, 16)] = acc1
            return 0

        lax.fori_loop(0, C // 16, acc_g, 0, unroll=False)


def _tec_body(x_hbm, tbl_hbm, out_hbm,
              x_v, xi0, xi1, i0a, i1a, i0b, i1b,
              f0a, f1a, f0b, f1b, oa, ob,
              sga, sgb, sxa, sxb, soa, sob):
    wid = lax.axis_index("s") * NC + lax.axis_index("c")
    pbase = wid * P_PER_W
    iota = lax.iota(jnp.int32, 16)
    iota3 = iota * jnp.int32(3)

    xidx_refs = (xi0, xi1)
    idx_refs = ((i0a, i1a), (i0b, i1b))
    feat_refs = ((f0a, f1a), (f0b, f1b))
    out_refs = (oa, ob)
    gsems = (sga, sgb)
    xsems = (sxa, sxb)
    osems = (soa, sob)
    XSZ = 3 * C

    def x_fire(par, ci):
        base = pbase + ci * C
        xi_v = xidx_refs[par]
        for d in range(3):
            def xg(g, _, d=d):
                xi_v[pl.ds(d * C + g * 16, 16)] = (
                    iota3 + jnp.int32(d) + 3 * (base + g * 16))
                return 0
            lax.fori_loop(0, C // 16, xg, 0, unroll=False)
        pltpu.make_async_copy(
            x_hbm.at[xi_v], x_v.at[pl.ds((ci % 4) * XSZ, XSZ)], xsems[par]
        ).start()

    def x_wait(par, ci):
        pltpu.make_async_copy(
            x_hbm.at[xidx_refs[par]], x_v.at[pl.ds((ci % 4) * XSZ, XSZ)],
            xsems[par],
        ).wait()

    def feat_fire(par, ci):
        xb = (ci % 4) * XSZ
        i0, i1 = idx_refs[par]
        f0, f1 = feat_refs[par]
        _index_pass(x_v, xb, i0, i1)
        pltpu.make_async_copy(tbl_hbm.at[i0], f0, gsems[par]).start()
        pltpu.make_async_copy(tbl_hbm.at[i1], f1, gsems[par]).start()

    def feat_drain(par):
        i0, i1 = idx_refs[par]
        f0, f1 = feat_refs[par]
        pltpu.make_async_copy(tbl_hbm.at[i0], f0, gsems[par]).wait()
        pltpu.make_async_copy(tbl_hbm.at[i1], f1, gsems[par]).wait()

    def out_start(par, ci):
        tc = (pbase // C) + ci
        o_v = out_refs[par]
        for j in range(NF):
            off = (j // 8) * (8 * N_POINTS) + tc * 1024 + (j % 8) * 128
            pltpu.make_async_copy(
                o_v.at[pl.ds(j * C, C)], out_hbm.at[pl.ds(off, C)], osems[par]
            ).start()

    def out_wait(par, ci):
        tc = (pbase // C) + ci
        o_v = out_refs[par]
        for j in range(NF):
            off = (j // 8) * (8 * N_POINTS) + tc * 1024 + (j % 8) * 128
            pltpu.make_async_copy(
                o_v.at[pl.ds(j * C, C)], out_hbm.at[pl.ds(off, C)], osems[par]
            ).wait()

    x_fire(0, 0)
    x_wait(0, 0)
    x_fire(1, 1)
    feat_fire(0, 0)

    def chunk_pair(cp, _):
        for par in (0, 1):
            ci = cp * 2 + par

            @pl.when(ci + 2 < N_CHUNKS)
            def _():
                x_fire(par, ci + 2)

            @pl.when(ci + 1 < N_CHUNKS)
            def _():
                x_wait(1 - par, ci + 1)
                feat_fire(1 - par, ci + 1)

            feat_drain(par)

            @pl.when(ci >= 2)
            def _():
                out_wait(par, ci - 2)

            _acc_pass(x_v, (ci % 4) * XSZ, *feat_refs[par], out_refs[par])
            out_start(par, ci)
        return 0

    lax.fori_loop(0, N_CHUNKS // 2, chunk_pair, 0, unroll=False)

    for par in (0, 1):
        out_wait(par, N_CHUNKS - 2 + par)


@jax.jit
def kernel(x, table, mask):
    del mask
    x_flat = x.reshape(3 * N_POINTS)
    tbl = (table.reshape(N_LEVELS, T // 128, 128, F_PER)
           .transpose(0, 1, 3, 2)
           .reshape(N_LEVELS * T * F_PER))

    mesh = plsc.VectorSubcoreMesh(
        core_axis_name="c", subcore_axis_name="s", num_cores=NC, num_subcores=NS
    )
    f = pl.kernel(
        _tec_body,
        out_type=jax.ShapeDtypeStruct((N_POINTS * NF,), jnp.float32),
        mesh=mesh,
        scratch_types=[
            pltpu.VMEM((4 * 3 * C,), jnp.float32),
            pltpu.VMEM((3 * C,), jnp.int32),
            pltpu.VMEM((3 * C,), jnp.int32),
            pltpu.VMEM((E_PER_F,), jnp.int32),
            pltpu.VMEM((E_PER_F,), jnp.int32),
            pltpu.VMEM((E_PER_F,), jnp.int32),
            pltpu.VMEM((E_PER_F,), jnp.int32),
            pltpu.VMEM((E_PER_F,), jnp.float32),
            pltpu.VMEM((E_PER_F,), jnp.float32),
            pltpu.VMEM((E_PER_F,), jnp.float32),
            pltpu.VMEM((E_PER_F,), jnp.float32),
            pltpu.VMEM((E_OUT,), jnp.float32),
            pltpu.VMEM((E_OUT,), jnp.float32),
            pltpu.SemaphoreType.DMA,
            pltpu.SemaphoreType.DMA,
            pltpu.SemaphoreType.DMA,
            pltpu.SemaphoreType.DMA,
            pltpu.SemaphoreType.DMA,
            pltpu.SemaphoreType.DMA,
        ],
    )
    flat = f(x_flat, tbl)
    return (flat.reshape(NF // 8, N_POINTS // 128, 8, 128)
            .transpose(1, 3, 0, 2)
            .reshape(N_POINTS, NF))


# speedup vs baseline: 1.0009x; 1.0009x over previous
import jax
import jax.numpy as jnp
import numpy as np
from jax import lax
from jax.experimental import pallas as pl
from jax.experimental.pallas import tpu as pltpu
from jax.experimental.pallas import tpu_sc as plsc

N_LEVELS = 12
F_PER = 2
LOG2_T = 19
T = 1 << LOG2_T
BASE_RES = 16
PER_LEVEL_SCALE = 2.0
N_POINTS = 524288

P1 = int(np.uint32(2654435761).view(np.int32))
P2 = int(np.uint32(805459861).view(np.int32))

NC, NS = 2, 16
NW = NC * NS
P_PER_W = N_POINTS // NW
C = 128
N_CHUNKS = P_PER_W // C
NF = N_LEVELS * F_PER

_RES = [int(np.ceil(BASE_RES * (PER_LEVEL_SCALE ** l))) for l in range(N_LEVELS)]
_DENSE = [(r + 1) ** 3 <= T for r in _RES]

N_SPM_LEVELS = 3
_ENTRIES = [min((r + 1) ** 3, T) for r in _RES]
_SPM_LEN = [2 * -(-_ENTRIES[l] // 128) * 128 for l in range(N_SPM_LEVELS)]
_SPM_BASE = [sum(_SPM_LEN[:l]) for l in range(N_SPM_LEVELS)]
SPM_SZ = sum(_SPM_LEN)

ES_PER_F = N_SPM_LEVELS * 8 * C
EH_PER_F = (N_LEVELS - N_SPM_LEVELS) * 8 * C


def _corner_offsets(l, ix, iy, iz):
    res = _RES[l]
    if _DENSE[l]:
        s = res + 1
        y0 = iy * jnp.int32(s)
        y1 = y0 + jnp.int32(s)
        z0 = iz * jnp.int32(s * s)
        z1 = z0 + jnp.int32(s * s)
        x1 = ix + jnp.int32(1)
        return [((x1 if (c & 1) else ix)
                 + (y1 if (c & 2) else y0)
                 + (z1 if (c & 4) else z0)) for c in range(8)]
    y0 = iy * jnp.int32(P1)
    y1 = y0 + jnp.int32(P1)
    z0 = iz * jnp.int32(P2)
    z1 = z0 + jnp.int32(P2)
    x1 = ix + jnp.int32(1)
    return [(((x1 if (c & 1) else ix)
              ^ (y1 if (c & 2) else y0)
              ^ (z1 if (c & 4) else z0)) & jnp.int32(T - 1)) for c in range(8)]


def _index_pass(x_v, xb, is0, ih0):
    for l in range(N_LEVELS):
        res = _RES[l]
        spm = l < N_SPM_LEVELS
        lbase = l * (2 * T)
        i0_v = is0 if spm else ih0
        lrow = l if spm else l - N_SPM_LEVELS

        def idx_g(g, _, l=l, res=res, lbase=lbase, i0_v=i0_v,
                  lrow=lrow):
            px = x_v[pl.ds(xb + 0 * C + g * 16, 16)] * jnp.float32(res)
            py = x_v[pl.ds(xb + 1 * C + g * 16, 16)] * jnp.float32(res)
            pz = x_v[pl.ds(xb + 2 * C + g * 16, 16)] * jnp.float32(res)
            ix = px.astype(jnp.int32)
            iy = py.astype(jnp.int32)
            iz = pz.astype(jnp.int32)
            for c, h in enumerate(_corner_offsets(l, ix, iy, iz)):
                e = h + (h & jnp.int32(-128)) + jnp.int32(lbase)
                off = (lrow * 8 + c) * C
                i0_v[pl.ds(off + g * 16, 16)] = e
            return 0

        lax.fori_loop(0, C // 16, idx_g, 0, unroll=False)


def _acc_pass(x_v, xb, fs0, fs1, fh0, fh1, out_v):
    for l in range(N_LEVELS):
        res = _RES[l]
        spm = l < N_SPM_LEVELS
        f0_v, f1_v = (fs0, fs1) if spm else (fh0, fh1)
        lrow = l if spm else l - N_SPM_LEVELS

        def acc_g(g, _, l=l, res=res, f0_v=f0_v, f1_v=f1_v, lrow=lrow):
            px = x_v[pl.ds(xb + 0 * C + g * 16, 16)] * jnp.float32(res)
            py = x_v[pl.ds(xb + 1 * C + g * 16, 16)] * jnp.float32(res)
            pz = x_v[pl.ds(xb + 2 * C + g * 16, 16)] * jnp.float32(res)
            wx = px - px.astype(jnp.int32).astype(jnp.float32)
            wy = py - py.astype(jnp.int32).astype(jnp.float32)
            wz = pz - pz.astype(jnp.int32).astype(jnp.float32)
            ox = jnp.float32(1.0) - wx
            oy = jnp.float32(1.0) - wy
            oz = jnp.float32(1.0) - wz
            wxy = (ox * oy, wx * oy, ox * wy, wx * wy)
            w8 = [wxy[c & 3] * (wz if (c & 4) else oz) for c in range(8)]
            base = (lrow * 8) * C + g * 16
            t0 = [None] * 8
            t1 = [None] * 8
            for c in range(8):
                f0 = f0_v[pl.ds(base + c * C, 16)]
                f1 = f1_v[pl.ds(base + c * C, 16)]
                t0[c] = f0 * w8[c]
                t1[c] = f1 * w8[c]
            acc0 = ((t0[0] + t0[1]) + (t0[2] + t0[3])) + (
                (t0[4] + t0[5]) + (t0[6] + t0[7]))
            acc1 = ((t1[0] + t1[1]) + (t1[2] + t1[3])) + (
                (t1[4] + t1[5]) + (t1[6] + t1[7]))
            out_v[pl.ds((2 * l) * C + g * 16, 16)] = acc0
            out_v[pl.ds((2 * l + 1) * C + g * 16, 16)] = acc1
            return 0

        lax.fori_loop(0, C // 16, acc_g, 0, unroll=False)


def _tec_body(x_hbm, tbl_hbm, out_hbm,
              x_v, xi0, xi1,
              is0a, ih0a, is0b, ih0b,
              fs0a, fs1a, fh0a, fh1a, fs0b, fs1b, fh0b, fh1b,
              oa, ob,
              sga, sgb, sxa, sxb, soa, sob):
    wid = lax.axis_index("s") * NC + lax.axis_index("c")
    pbase = wid * P_PER_W
    iota = lax.iota(jnp.int32, 16)
    iota3 = iota * jnp.int32(3)

    xidx_refs = (xi0, xi1)
    sidx_refs = (is0a, is0b)
    hidx_refs = (ih0a, ih0b)
    tbl1 = tbl_hbm.at[pl.ds(128, N_LEVELS * T * F_PER - 128)]
    sfeat_refs = ((fs0a, fs1a), (fs0b, fs1b))
    hfeat_refs = ((fh0a, fh1a), (fh0b, fh1b))
    out_refs = (oa, ob)
    gsems = (sga, sgb)
    xsems = (sxa, sxb)
    osems = (soa, sob)
    XSZ = 3 * C

    def x_fire(par, ci):
        base = pbase + ci * C
        xi_v = xidx_refs[par]
        for d in range(3):
            def xg(g, _, d=d):
                xi_v[pl.ds(d * C + g * 16, 16)] = (
                    iota3 + jnp.int32(d) + 3 * (base + g * 16))
                return 0
            lax.fori_loop(0, C // 16, xg, 0, unroll=False)
        pltpu.make_async_copy(
            x_hbm.at[xi_v], x_v.at[pl.ds((ci % 4) * XSZ, XSZ)], xsems[par]
        ).start()

    def x_wait(par, ci):
        pltpu.make_async_copy(
            x_hbm.at[xidx_refs[par]], x_v.at[pl.ds((ci % 4) * XSZ, XSZ)],
            xsems[par],
        ).wait()

    def feat_fire(par, ci):
        xb = (ci % 4) * XSZ
        is0 = sidx_refs[par]
        ih0 = hidx_refs[par]
        fs0, fs1 = sfeat_refs[par]
        fh0, fh1 = hfeat_refs[par]
        _index_pass(x_v, xb, is0, ih0)
        pltpu.make_async_copy(tbl_hbm.at[is0], fs0, gsems[par]).start()
        pltpu.make_async_copy(tbl1.at[is0], fs1, gsems[par]).start()
        pltpu.make_async_copy(tbl_hbm.at[ih0], fh0, gsems[par]).start()
        pltpu.make_async_copy(tbl1.at[ih0], fh1, gsems[par]).start()

    def feat_drain(par):
        is0 = sidx_refs[par]
        ih0 = hidx_refs[par]
        fs0, fs1 = sfeat_refs[par]
        fh0, fh1 = hfeat_refs[par]
        pltpu.make_async_copy(tbl_hbm.at[is0], fs0, gsems[par]).wait()
        pltpu.make_async_copy(tbl1.at[is0], fs1, gsems[par]).wait()
        pltpu.make_async_copy(tbl_hbm.at[ih0], fh0, gsems[par]).wait()
        pltpu.make_async_copy(tbl1.at[ih0], fh1, gsems[par]).wait()

    def out_start(par, ci):
        tc = (pbase // C) + ci
        o_v = out_refs[par]
        for j in range(NF):
            off = (j // 8) * (8 * N_POINTS) + tc * 1024 + (j % 8) * 128
            pltpu.make_async_copy(
                o_v.at[pl.ds(j * C, C)], out_hbm.at[pl.ds(off, C)], osems[par]
            ).start()

    def out_wait(par, ci):
        tc = (pbase // C) + ci
        o_v = out_refs[par]
        for j in range(NF):
            off = (j // 8) * (8 * N_POINTS) + tc * 1024 + (j % 8) * 128
            pltpu.make_async_copy(
                o_v.at[pl.ds(j * C, C)], out_hbm.at[pl.ds(off, C)], osems[par]
            ).wait()

    x_fire(0, 0)
    x_wait(0, 0)
    x_fire(1, 1)
    feat_fire(0, 0)

    def chunk_pair(cp, _):
        for par in (0, 1):
            ci = cp * 2 + par

            @pl.when(ci + 2 < N_CHUNKS)
            def _():
                x_fire(par, ci + 2)

            @pl.when(ci + 1 < N_CHUNKS)
            def _():
                x_wait(1 - par, ci + 1)
                feat_fire(1 - par, ci + 1)

            feat_drain(par)

            @pl.when(ci >= 2)
            def _():
                out_wait(par, ci - 2)

            _acc_pass(x_v, (ci % 4) * XSZ, *sfeat_refs[par], *hfeat_refs[par],
                      out_refs[par])
            out_start(par, ci)
        return 0

    lax.fori_loop(0, N_CHUNKS // 2, chunk_pair, 0, unroll=False)

    for par in (0, 1):
        out_wait(par, N_CHUNKS - 2 + par)


@jax.jit
def kernel(x, table, mask):
    del mask
    x_flat = x.reshape(3 * N_POINTS)
    tbl = (table.reshape(N_LEVELS, T // 128, 128, F_PER)
           .transpose(0, 1, 3, 2)
           .reshape(N_LEVELS * T * F_PER))

    mesh = plsc.VectorSubcoreMesh(
        core_axis_name="c", subcore_axis_name="s", num_cores=NC, num_subcores=NS
    )
    f = pl.kernel(
        _tec_body,
        out_type=jax.ShapeDtypeStruct((N_POINTS * NF,), jnp.float32),
        mesh=mesh,
        scratch_types=[
            pltpu.VMEM((4 * 3 * C,), jnp.float32),
            pltpu.VMEM((3 * C,), jnp.int32),
            pltpu.VMEM((3 * C,), jnp.int32),
            pltpu.VMEM((ES_PER_F,), jnp.int32),
            pltpu.VMEM((EH_PER_F,), jnp.int32),
            pltpu.VMEM((ES_PER_F,), jnp.int32),
            pltpu.VMEM((EH_PER_F,), jnp.int32),
            pltpu.VMEM((ES_PER_F,), jnp.float32),
            pltpu.VMEM((ES_PER_F,), jnp.float32),
            pltpu.VMEM((EH_PER_F,), jnp.float32),
            pltpu.VMEM((EH_PER_F,), jnp.float32),
            pltpu.VMEM((ES_PER_F,), jnp.float32),
            pltpu.VMEM((ES_PER_F,), jnp.float32),
            pltpu.VMEM((EH_PER_F,), jnp.float32),
            pltpu.VMEM((EH_PER_F,), jnp.float32),
            pltpu.VMEM((NF * C,), jnp.float32),
            pltpu.VMEM((NF * C,), jnp.float32),
            pltpu.SemaphoreType.DMA,
            pltpu.SemaphoreType.DMA,
            pltpu.SemaphoreType.DMA,
            pltpu.SemaphoreType.DMA,
            pltpu.SemaphoreType.DMA,
            pltpu.SemaphoreType.DMA,
        ],
    )
    flat = f(x_flat, tbl)
    return (flat.reshape(NF // 8, N_POINTS // 128, 8, 128)
            .transpose(1, 3, 0, 2)
            .reshape(N_POINTS, NF))


# speedup vs baseline: 1.0009x; 1.0001x over previous
import jax
import jax.numpy as jnp
import numpy as np
from jax import lax
from jax.experimental import pallas as pl
from jax.experimental.pallas import tpu as pltpu
from jax.experimental.pallas import tpu_sc as plsc

N_LEVELS = 12
F_PER = 2
LOG2_T = 19
T = 1 << LOG2_T
BASE_RES = 16
PER_LEVEL_SCALE = 2.0
N_POINTS = 524288

P1 = int(np.uint32(2654435761).view(np.int32))
P2 = int(np.uint32(805459861).view(np.int32))

NC, NS = 2, 16
NW = NC * NS
P_PER_W = N_POINTS // NW
C = 128
N_CHUNKS = P_PER_W // C
NF = N_LEVELS * F_PER

_RES = [int(np.ceil(BASE_RES * (PER_LEVEL_SCALE ** l))) for l in range(N_LEVELS)]
_DENSE = [(r + 1) ** 3 <= T for r in _RES]

N_LO_LEVELS = 3
EL_PER_F = N_LO_LEVELS * 8 * C
EH_PER_F = (N_LEVELS - N_LO_LEVELS) * 8 * C


def _corner_offsets(l, ix, iy, iz):
    res = _RES[l]
    if _DENSE[l]:
        s = res + 1
        y0 = iy * jnp.int32(s)
        y1 = y0 + jnp.int32(s)
        z0 = iz * jnp.int32(s * s)
        z1 = z0 + jnp.int32(s * s)
        x1 = ix + jnp.int32(1)
        return [((x1 if (c & 1) else ix)
                 + (y1 if (c & 2) else y0)
                 + (z1 if (c & 4) else z0)) for c in range(8)]
    y0 = iy * jnp.int32(P1)
    y1 = y0 + jnp.int32(P1)
    z0 = iz * jnp.int32(P2)
    z1 = z0 + jnp.int32(P2)
    x1 = ix + jnp.int32(1)
    return [(((x1 if (c & 1) else ix)
              ^ (y1 if (c & 2) else y0)
              ^ (z1 if (c & 4) else z0)) & jnp.int32(T - 1)) for c in range(8)]


def _index_pass(x_v, xb, ilo, ihi):
    for l in range(N_LEVELS):
        res = _RES[l]
        lo = l < N_LO_LEVELS
        lbase = l * (2 * T)
        i0_v = ilo if lo else ihi
        lrow = l if lo else l - N_LO_LEVELS

        def idx_g(g, _, l=l, res=res, lbase=lbase, i0_v=i0_v,
                  lrow=lrow):
            px = x_v[pl.ds(xb + 0 * C + g * 16, 16)] * jnp.float32(res)
            py = x_v[pl.ds(xb + 1 * C + g * 16, 16)] * jnp.float32(res)
            pz = x_v[pl.ds(xb + 2 * C + g * 16, 16)] * jnp.float32(res)
            ix = px.astype(jnp.int32)
            iy = py.astype(jnp.int32)
            iz = pz.astype(jnp.int32)
            for c, h in enumerate(_corner_offsets(l, ix, iy, iz)):
                e = h + (h & jnp.int32(-128)) + jnp.int32(lbase)
                off = (lrow * 8 + c) * C
                i0_v[pl.ds(off + g * 16, 16)] = e
            return 0

        lax.fori_loop(0, C // 16, idx_g, 0, unroll=False)


def _acc_pass(x_v, xb, flo0, flo1, fhi0, fhi1, out_v):
    for l in range(N_LEVELS):
        res = _RES[l]
        lo = l < N_LO_LEVELS
        f0_v, f1_v = (flo0, flo1) if lo else (fhi0, fhi1)
        lrow = l if lo else l - N_LO_LEVELS

        def acc_g(g, _, l=l, res=res, f0_v=f0_v, f1_v=f1_v, lrow=lrow):
            px = x_v[pl.ds(xb + 0 * C + g * 16, 16)] * jnp.float32(res)
            py = x_v[pl.ds(xb + 1 * C + g * 16, 16)] * jnp.float32(res)
            pz = x_v[pl.ds(xb + 2 * C + g * 16, 16)] * jnp.float32(res)
            wx = px - px.astype(jnp.int32).astype(jnp.float32)
            wy = py - py.astype(jnp.int32).astype(jnp.float32)
            wz = pz - pz.astype(jnp.int32).astype(jnp.float32)
            ox = jnp.float32(1.0) - wx
            oy = jnp.float32(1.0) - wy
            oz = jnp.float32(1.0) - wz
            wxy = (ox * oy, wx * oy, ox * wy, wx * wy)
            w8 = [wxy[c & 3] * (wz if (c & 4) else oz) for c in range(8)]
            base = (lrow * 8) * C + g * 16
            t0 = [None] * 8
            t1 = [None] * 8
            for c in range(8):
                f0 = f0_v[pl.ds(base + c * C, 16)]
                f1 = f1_v[pl.ds(base + c * C, 16)]
                t0[c] = f0 * w8[c]
                t1[c] = f1 * w8[c]
            acc0 = ((t0[0] + t0[1]) + (t0[2] + t0[3])) + (
                (t0[4] + t0[5]) + (t0[6] + t0[7]))
            acc1 = ((t1[0] + t1[1]) + (t1[2] + t1[3])) + (
                (t1[4] + t1[5]) + (t1[6] + t1[7]))
            out_v[pl.ds((2 * l) * C + g * 16, 16)] = acc0
            out_v[pl.ds((2 * l + 1) * C + g * 16, 16)] = acc1
            return 0

        lax.fori_loop(0, C // 16, acc_g, 0, unroll=False)


def _tec_body(x_hbm, tbl_hbm, out_hbm,
              x_v, xi0, xi1,
              ilo_a, ihi_a, ilo_b, ihi_b,
              flo0a, flo1a, fhi0a, fhi1a, flo0b, flo1b, fhi0b, fhi1b,
              oa, ob,
              sga, sgb, sxa, sxb, soa, sob):
    wid = lax.axis_index("s") * NC + lax.axis_index("c")
    pbase = wid * P_PER_W
    iota = lax.iota(jnp.int32, 16)
    iota3 = iota * jnp.int32(3)

    xidx_refs = (xi0, xi1)
    lo_idx_refs = (ilo_a, ilo_b)
    hi_idx_refs = (ihi_a, ihi_b)
    tbl1 = tbl_hbm.at[pl.ds(128, N_LEVELS * T * F_PER - 128)]
    lo_feat_refs = ((flo0a, flo1a), (flo0b, flo1b))
    hi_feat_refs = ((fhi0a, fhi1a), (fhi0b, fhi1b))
    out_refs = (oa, ob)
    gsems = (sga, sgb)
    xsems = (sxa, sxb)
    osems = (soa, sob)
    XSZ = 3 * C

    def x_fire(par, ci):
        base = pbase + ci * C
        xi_v = xidx_refs[par]
        for d in range(3):
            def xg(g, _, d=d):
                xi_v[pl.ds(d * C + g * 16, 16)] = (
                    iota3 + jnp.int32(d) + 3 * (base + g * 16))
                return 0
            lax.fori_loop(0, C // 16, xg, 0, unroll=False)
        pltpu.make_async_copy(
            x_hbm.at[xi_v], x_v.at[pl.ds((ci % 4) * XSZ, XSZ)], xsems[par]
        ).start()

    def x_wait(par, ci):
        pltpu.make_async_copy(
            x_hbm.at[xidx_refs[par]], x_v.at[pl.ds((ci % 4) * XSZ, XSZ)],
            xsems[par],
        ).wait()

    def feat_fire(par, ci):
        xb = (ci % 4) * XSZ
        ilo = lo_idx_refs[par]
        ihi = hi_idx_refs[par]
        flo0, flo1 = lo_feat_refs[par]
        fhi0, fhi1 = hi_feat_refs[par]
        _index_pass(x_v, xb, ilo, ihi)
        pltpu.make_async_copy(tbl_hbm.at[ilo], flo0, gsems[par]).start()
        pltpu.make_async_copy(tbl1.at[ilo], flo1, gsems[par]).start()
        pltpu.make_async_copy(tbl_hbm.at[ihi], fhi0, gsems[par]).start()
        pltpu.make_async_copy(tbl1.at[ihi], fhi1, gsems[par]).start()

    def feat_drain(par):
        ilo = lo_idx_refs[par]
        ihi = hi_idx_refs[par]
        flo0, flo1 = lo_feat_refs[par]
        fhi0, fhi1 = hi_feat_refs[par]
        pltpu.make_async_copy(tbl_hbm.at[ilo], flo0, gsems[par]).wait()
        pltpu.make_async_copy(tbl1.at[ilo], flo1, gsems[par]).wait()
        pltpu.make_async_copy(tbl_hbm.at[ihi], fhi0, gsems[par]).wait()
        pltpu.make_async_copy(tbl1.at[ihi], fhi1, gsems[par]).wait()

    def out_start(par, ci):
        tc = (pbase // C) + ci
        o_v = out_refs[par]
        for j in range(NF):
            off = (j // 8) * (8 * N_POINTS) + tc * 1024 + (j % 8) * 128
            pltpu.make_async_copy(
                o_v.at[pl.ds(j * C, C)], out_hbm.at[pl.ds(off, C)], osems[par]
            ).start()

    def out_wait(par, ci):
        tc = (pbase // C) + ci
        o_v = out_refs[par]
        for j in range(NF):
            off = (j // 8) * (8 * N_POINTS) + tc * 1024 + (j % 8) * 128
            pltpu.make_async_copy(
                o_v.at[pl.ds(j * C, C)], out_hbm.at[pl.ds(off, C)], osems[par]
            ).wait()

    x_fire(0, 0)
    x_wait(0, 0)
    x_fire(1, 1)
    feat_fire(0, 0)

    def chunk_pair(cp, _):
        for par in (0, 1):
            ci = cp * 2 + par

            @pl.when(ci + 2 < N_CHUNKS)
            def _():
                x_fire(par, ci + 2)

            @pl.when(ci + 1 < N_CHUNKS)
            def _():
                x_wait(1 - par, ci + 1)
                feat_fire(1 - par, ci + 1)

            feat_drain(par)

            @pl.when(ci >= 2)
            def _():
                out_wait(par, ci - 2)

            _acc_pass(x_v, (ci % 4) * XSZ, *lo_feat_refs[par], *hi_feat_refs[par],
                      out_refs[par])
            out_start(par, ci)
        return 0

    lax.fori_loop(0, N_CHUNKS // 2, chunk_pair, 0, unroll=False)

    for par in (0, 1):
        out_wait(par, N_CHUNKS - 2 + par)


@jax.jit
def kernel(x, table, mask):
    del mask
    x_flat = x.reshape(3 * N_POINTS)
    tbl = (table.reshape(N_LEVELS, T // 128, 128, F_PER)
           .transpose(0, 1, 3, 2)
           .reshape(N_LEVELS * T * F_PER))

    mesh = plsc.VectorSubcoreMesh(
        core_axis_name="c", subcore_axis_name="s", num_cores=NC, num_subcores=NS
    )
    f = pl.kernel(
        _tec_body,
        out_type=jax.ShapeDtypeStruct((N_POINTS * NF,), jnp.float32),
        mesh=mesh,
        scratch_types=[
            pltpu.VMEM((4 * 3 * C,), jnp.float32),
            pltpu.VMEM((3 * C,), jnp.int32),
            pltpu.VMEM((3 * C,), jnp.int32),
            pltpu.VMEM((EL_PER_F,), jnp.int32),
            pltpu.VMEM((EH_PER_F,), jnp.int32),
            pltpu.VMEM((EL_PER_F,), jnp.int32),
            pltpu.VMEM((EH_PER_F,), jnp.int32),
            pltpu.VMEM((EL_PER_F,), jnp.float32),
            pltpu.VMEM((EL_PER_F,), jnp.float32),
            pltpu.VMEM((EH_PER_F,), jnp.float32),
            pltpu.VMEM((EH_PER_F,), jnp.float32),
            pltpu.VMEM((EL_PER_F,), jnp.float32),
            pltpu.VMEM((EL_PER_F,), jnp.float32),
            pltpu.VMEM((EH_PER_F,), jnp.float32),
            pltpu.VMEM((EH_PER_F,), jnp.float32),
            pltpu.VMEM((NF * C,), jnp.float32),
            pltpu.VMEM((NF * C,), jnp.float32),
            pltpu.SemaphoreType.DMA,
            pltpu.SemaphoreType.DMA,
            pltpu.SemaphoreType.DMA,
            pltpu.SemaphoreType.DMA,
            pltpu.SemaphoreType.DMA,
            pltpu.SemaphoreType.DMA,
        ],
    )
    flat = f(x_flat, tbl)
    return (flat.reshape(NF // 8, N_POINTS // 128, 8, 128)
            .transpose(1, 3, 0, 2)
            .reshape(N_POINTS, NF))
